# baseline (speedup 1.0000x reference)
import functools
import math

import jax
import jax.numpy as jnp
import numpy as np
from jax import lax
from jax.experimental import pallas as pl
from jax.experimental.pallas import tpu as pltpu
from jax.experimental.pallas import tpu_sc as plsc

N = 10000
E = 320000
D = 128
A = 16
B = 8
H = 64
EHALF = E // 2

_z = np.linspace(-10.0, 10.0, 200001)
_pdf = np.exp(-0.5 * _z ** 2) / np.sqrt(2.0 * np.pi)
_silu_np = _z / (1.0 + np.exp(-_z))
_NSILU_CST = float(1.0 / np.sqrt(np.trapz(_silu_np ** 2 * _pdf, _z)))

_PREC = jax.lax.Precision.DEFAULT


def _nsilu(v):
    return jax.nn.silu(v) * _NSILU_CST


_PARTS = (
    (0, 49152, 6144, 96),
    (49152, 98304, 6144, 96),
    (147456, 172544, 512, 96),
)


def _edge_mlp_body(embt_ref, eat_ref, wfc1_ref, wfc2_ref, out_ref):
    h = jax.lax.dot_general(
        embt_ref[...], wfc1_ref[...], (((0,), (0,)), ((), ())),
        precision=_PREC, preferred_element_type=jnp.float32,
    ) * (1.0 / math.sqrt(B))
    h = _nsilu(h)
    ea_col = jax.lax.dot_general(
        eat_ref[...], jnp.ones((1, 1), jnp.float32), (((0,), (0,)), ((), ())),
        precision=_PREC, preferred_element_type=jnp.float32,
    )
    h = h * ea_col
    w = jnp.dot(h, wfc2_ref[...], precision=_PREC,
                preferred_element_type=jnp.float32) * (1.0 / math.sqrt(H))
    out_ref[...] = w


def _edge_mlp_part(embt, eat, wfc1, wfc2, estart, esize, be):
    off = estart // be
    return pl.pallas_call(
        _edge_mlp_body,
        grid=(esize // be,),
        in_specs=[
            pl.BlockSpec((B, be), lambda i: (0, i + off)),
            pl.BlockSpec((1, be), lambda i: (0, i + off)),
            pl.BlockSpec((B, H), lambda i: (0, 0)),
            pl.BlockSpec((H, D), lambda i: (0, 0)),
        ],
        out_specs=pl.BlockSpec((be, D), lambda i: (i, 0)),
        out_shape=jax.ShapeDtypeStruct((esize, D), jnp.float32),
    )(embt, eat, wfc1, wfc2)


_BN1 = 2000


def _node_lin_body(x0_ref, w_ref, out_ref):
    out_ref[...] = jnp.dot(x0_ref[...], w_ref[...], precision=_PREC,
                           preferred_element_type=jnp.float32) * (1.0 / math.sqrt(D))


def _node_linear(x0, wlin1):
    grid = (N // _BN1,)
    return pl.pallas_call(
        _node_lin_body,
        grid=grid,
        in_specs=[
            pl.BlockSpec((_BN1, D), lambda i: (i, 0)),
            pl.BlockSpec((D, D), lambda i: (0, 0)),
        ],
        out_specs=pl.BlockSpec((_BN1, D), lambda i: (i, 0)),
        out_shape=jax.ShapeDtypeStruct((N, D), jnp.float32),
    )(x0, wlin1)


def _split_idx_body(idx_ref, ei_ref, ej_ref):
    idx = idx_ref[...]
    ei_ref[...] = idx[0]
    ej_ref[...] = idx[1]


def _split_idx(idx):
    return pl.pallas_call(
        _split_idx_body,
        in_specs=[pl.BlockSpec((2, E), lambda: (0, 0))],
        out_specs=[
            pl.BlockSpec((E,), lambda: (0,)),
            pl.BlockSpec((E,), lambda: (0,)),
        ],
        out_shape=[
            jax.ShapeDtypeStruct((E,), jnp.int32),
            jax.ShapeDtypeStruct((E,), jnp.int32),
        ],
    )(idx)


_NC = 2
_NS = 16
_NW = _NC * _NS
_RPT = 624
_RTAIL = N - _RPT * _NS


def _sc_body_fn(ebase0, ept, ch):
    _EPT = ept
    _CH = ch
    _NFULL = _EPT // _CH
    _REM = _EPT - _NFULL * _CH
    assert _NFULL % 2 == 0 and _NFULL >= 4 and _CH % 8 == 0 and _REM % 8 == 0
    def _sc_body(x_hbm, wq_hbm, ei_hbm, ej_hbm, init_hbm, out_hbm,
                 acc, ei_v, ej_v, w_v, xg_v, ei_r, ej_r,
                 sem_in, sem_g, sem_s):
        cid = lax.axis_index("c")
        sid = lax.axis_index("s")
        pltpu.sync_copy(init_hbm.at[cid, pl.ds(sid * _RPT, _RPT)],
                        acc.at[pl.ds(sid * _RPT, _RPT)])

        @pl.when(sid == 0)
        def _seed_tail():
            pltpu.sync_copy(init_hbm.at[cid, pl.ds(_RPT * _NS, _RTAIL)],
                            acc.at[pl.ds(_RPT * _NS, _RTAIL)])
        plsc.subcore_barrier()

        ebase = ebase0 + (cid * _NS + sid) * _EPT

        def start_in(k, b):
            base = pl.multiple_of(ebase + k * _CH, 8)
            pltpu.async_copy(ei_hbm.at[pl.ds(base, _CH)], ei_v.at[b],
                             sem_in.at[b])
            pltpu.async_copy(ej_hbm.at[pl.ds(base, _CH)], ej_v.at[b],
                             sem_in.at[b])
            pltpu.async_copy(wq_hbm.at[pl.ds(base - ebase0, _CH)], w_v.at[b],
                             sem_in.at[b])

        def wait_in(k, b):
            base = pl.multiple_of(ebase + k * _CH, 8)
            pltpu.make_async_copy(ei_hbm.at[pl.ds(base, _CH)], ei_v.at[b],
                                  sem_in.at[b]).wait()
            pltpu.make_async_copy(ej_hbm.at[pl.ds(base, _CH)], ej_v.at[b],
                                  sem_in.at[b]).wait()
            pltpu.make_async_copy(wq_hbm.at[pl.ds(base - ebase0, _CH)],
                                  w_v.at[b], sem_in.at[b]).wait()

        def start_gather(b):
            pltpu.async_copy(x_hbm.at[ei_v.at[b]], xg_v.at[b], sem_g.at[b])

        def wait_gather(b):
            pltpu.make_async_copy(x_hbm.at[ei_v.at[b]], xg_v.at[b],
                                  sem_g.at[b]).wait()

        def start_scatter(b):
            pltpu.async_copy(xg_v.at[b], acc.at[ej_v.at[b]], sem_s.at[b],
                             add=True)

        def wait_scatter(b):
            pltpu.make_async_copy(xg_v.at[b], acc.at[ej_v.at[b]],
                                  sem_s.at[b]).wait()

        def compute(b):
            def rows(i, c2):
                i0 = i * 2
                i1 = i0 + 1
                for j in range(D // 16):
                    sl = pl.ds(j * 16, 16)
                    xg_v[b, i0, sl] = xg_v[b, i0, sl] * w_v[b, i0, sl]
                for j in range(D // 16):
                    sl = pl.ds(j * 16, 16)
                    xg_v[b, i1, sl] = xg_v[b, i1, sl] * w_v[b, i1, sl]
                return c2
            lax.fori_loop(0, _CH // 2, rows, 0)

        def body(k, b, first, last):
            o = 1 - b
            if not last:
                wait_in(k + 1, o)
            if not first:
                wait_scatter(o)
            if not last:
                start_gather(o)
            wait_gather(b)
            compute(b)
            start_scatter(b)
            if not last:
                start_in(k + 2, b)

        start_in(0, 0)
        start_in(1, 1)
        wait_in(0, 0)
        start_gather(0)
        body(0, 0, first=True, last=False)
        body(1, 1, first=False, last=False)

        def pair(m, carry):
            k = m * 2
            body(k, 0, first=False, last=False)
            body(k + 1, 1, first=False, last=False)
            return carry

        lax.fori_loop(1, _NFULL // 2 - 1, pair, 0)

        kl = _NFULL - 2
        wait_in(kl + 1, 1)
        wait_scatter(1)
        start_gather(1)
        wait_gather(0)
        compute(0)
        start_scatter(0)
        body(kl + 1, 1, first=False, last=True)
        wait_scatter(1)

        if _REM:
            rbase = pl.multiple_of(ebase + _NFULL * _CH, 8)
            pltpu.sync_copy(ei_hbm.at[pl.ds(rbase, _REM)], ei_r)
            pltpu.sync_copy(ej_hbm.at[pl.ds(rbase, _REM)], ej_r)
            w_r = w_v.at[0, pl.ds(0, _REM)]
            xg_r = xg_v.at[0, pl.ds(0, _REM)]
            pltpu.sync_copy(wq_hbm.at[pl.ds(rbase - ebase0, _REM)], w_r)
            pltpu.async_copy(x_hbm.at[ei_r], xg_r, sem_g.at[0]).wait()

            def rrow(i, c2):
                for j in range(D // 16):
                    sl = pl.ds(j * 16, 16)
                    xg_v[0, i, sl] = xg_v[0, i, sl] * w_v[0, i, sl]
                return c2
            lax.fori_loop(0, _REM, rrow, 0)
            pltpu.sync_copy(xg_r, acc.at[ej_r], add=True)

        plsc.subcore_barrier()
        pltpu.sync_copy(acc.at[pl.ds(sid * _RPT, _RPT)],
                        out_hbm.at[cid, pl.ds(sid * _RPT, _RPT)])

        @pl.when(sid == 0)
        def _dump_tail():
            pltpu.sync_copy(acc.at[pl.ds(_RPT * _NS, _RTAIL)],
                            out_hbm.at[cid, pl.ds(_RPT * _NS, _RTAIL)])

    return _sc_body


@functools.lru_cache(maxsize=2)
def _make_sc_kernel(ebase0, ept, ch):
    rem = ept - (ept // ch) * ch
    mesh = plsc.VectorSubcoreMesh(core_axis_name="c", subcore_axis_name="s",
                                  num_cores=_NC, num_subcores=_NS)
    return pl.kernel(
        _sc_body_fn(ebase0, ept, ch),
        out_type=jax.ShapeDtypeStruct((_NC, N, D), jnp.float32),
        mesh=mesh,
        scratch_types=[
            pltpu.VMEM_SHARED((N, D), jnp.float32),
            pltpu.VMEM((2, ch), jnp.int32),
            pltpu.VMEM((2, ch), jnp.int32),
            pltpu.VMEM((2, ch, D), jnp.float32),
            pltpu.VMEM((2, ch, D), jnp.float32),
            pltpu.VMEM((max(rem, 8),), jnp.int32),
            pltpu.VMEM((max(rem, 8),), jnp.int32),
            pltpu.SemaphoreType.DMA((2,)),
            pltpu.SemaphoreType.DMA((2,)),
            pltpu.SemaphoreType.DMA((2,)),
        ],
    )


_BN3 = 400


def _selfconn_body(x0_ref, attrs_ref, wsc2d_ref, out_ref):
    t = jnp.dot(x0_ref[...], wsc2d_ref[...], precision=_PREC,
                preferred_element_type=jnp.float32)
    attrs = attrs_ref[...]
    sc = t[:, 0:D] * attrs[:, 0:1]
    for v in range(1, A):
        sc = sc + t[:, v * D:(v + 1) * D] * attrs[:, v:v + 1]
    out_ref[...] = sc * (1.0 / math.sqrt(float(D * A)))


def _selfconn(x0, attrs, wsc2d):
    grid = (N // _BN3,)
    return pl.pallas_call(
        _selfconn_body,
        grid=grid,
        in_specs=[
            pl.BlockSpec((_BN3, D), lambda i: (i, 0)),
            pl.BlockSpec((_BN3, A), lambda i: (i, 0)),
            pl.BlockSpec((D, A * D), lambda i: (0, 0)),
        ],
        out_specs=pl.BlockSpec((_BN3, D), lambda i: (i, 0)),
        out_shape=jax.ShapeDtypeStruct((N, D), jnp.float32),
    )(x0, attrs, wsc2d)


def _final_body(agg2_ref, x0_ref, sc_ref, wlin2_ref, out_ref):
    agg = agg2_ref[0] + agg2_ref[1]
    z = jnp.dot(agg, wlin2_ref[...], precision=_PREC,
                preferred_element_type=jnp.float32) * (1.0 / math.sqrt(D))
    x0 = x0_ref[...]
    out_ref[...] = _nsilu(z + sc_ref[...]) + x0


def _final(agg2, x0, sc, wlin2):
    grid = (N // _BN3,)
    return pl.pallas_call(
        _final_body,
        grid=grid,
        in_specs=[
            pl.BlockSpec((_NC, _BN3, D), lambda i: (0, i, 0)),
            pl.BlockSpec((_BN3, D), lambda i: (i, 0)),
            pl.BlockSpec((_BN3, D), lambda i: (i, 0)),
            pl.BlockSpec((D, D), lambda i: (0, 0)),
        ],
        out_specs=pl.BlockSpec((_BN3, D), lambda i: (i, 0)),
        out_shape=jax.ShapeDtypeStruct((N, D), jnp.float32),
    )(agg2, x0, sc, wlin2)


def kernel(node_features, node_attrs, edge_attrs, edge_embedding, edge_index,
           W_lin1, W_fc1, W_fc2, W_lin2, W_sc):
    embt = edge_embedding.T
    eat = edge_attrs.T
    x = _node_linear(node_features, W_lin1)
    ei, ej = _split_idx(edge_index)
    agg = jnp.zeros((_NC, N, D), jnp.float32)
    wq = _edge_mlp_part(embt, eat, W_fc1, W_fc2, *_PARTS[0][:3])
    for p in range(len(_PARTS)):
        estart, esize, _, ch = _PARTS[p]
        nxt = None
        if p + 1 < len(_PARTS):
            nxt = _edge_mlp_part(embt, eat, W_fc1, W_fc2, *_PARTS[p + 1][:3])
        if p == len(_PARTS) - 1:
            sc = _selfconn(node_features, node_attrs, W_sc.reshape(D, A * D))
        agg = _make_sc_kernel(estart, esize // _NW, ch)(x, wq, ei, ej, agg)
        wq = nxt
    return _final(agg, node_features, sc, W_lin2)


# speedup vs baseline: 1.3276x; 1.3276x over previous
import functools
import math

import jax
import jax.numpy as jnp
import numpy as np
from jax import lax
from jax.experimental import pallas as pl
from jax.experimental.pallas import tpu as pltpu
from jax.experimental.pallas import tpu_sc as plsc

N = 10000
E = 320000
D = 128
A = 16
B = 8
H = 64
EHALF = E // 2

_z = np.linspace(-10.0, 10.0, 200001)
_pdf = np.exp(-0.5 * _z ** 2) / np.sqrt(2.0 * np.pi)
_silu_np = _z / (1.0 + np.exp(-_z))
_NSILU_CST = float(1.0 / np.sqrt(np.trapz(_silu_np ** 2 * _pdf, _z)))

_PREC = jax.lax.Precision.DEFAULT


def _nsilu(v):
    return jax.nn.silu(v) * _NSILU_CST


_PARTS = (
    (0, 115200, 5760, 72),
    (115200, 204800, 6400, 96),
)


def _edge_mlp_body(embt_ref, eat_ref, wfc1_ref, wfc2_ref, out_ref):
    h = jax.lax.dot_general(
        embt_ref[...], wfc1_ref[...], (((0,), (0,)), ((), ())),
        precision=_PREC, preferred_element_type=jnp.float32,
    ) * (1.0 / math.sqrt(B))
    h = _nsilu(h)
    ea_col = jax.lax.dot_general(
        eat_ref[...], jnp.ones((1, 1), jnp.float32), (((0,), (0,)), ((), ())),
        precision=_PREC, preferred_element_type=jnp.float32,
    )
    h = h * ea_col
    w = jnp.dot(h, wfc2_ref[...], precision=_PREC,
                preferred_element_type=jnp.float32) * (1.0 / math.sqrt(H))
    out_ref[...] = w


def _edge_mlp_part(embt, eat, wfc1, wfc2, estart, esize, be):
    off = estart // be
    return pl.pallas_call(
        _edge_mlp_body,
        grid=(esize // be,),
        in_specs=[
            pl.BlockSpec((B, be), lambda i: (0, i + off)),
            pl.BlockSpec((1, be), lambda i: (0, i + off)),
            pl.BlockSpec((B, H), lambda i: (0, 0)),
            pl.BlockSpec((H, D), lambda i: (0, 0)),
        ],
        out_specs=pl.BlockSpec((be, D), lambda i: (i, 0)),
        out_shape=jax.ShapeDtypeStruct((esize, D), jnp.float32),
    )(embt, eat, wfc1, wfc2)


_BN1 = 2000


def _node_lin_body(x0_ref, w_ref, out_ref):
    out_ref[...] = jnp.dot(x0_ref[...], w_ref[...], precision=_PREC,
                           preferred_element_type=jnp.float32) * (1.0 / math.sqrt(D))


def _node_linear(x0, wlin1):
    grid = (N // _BN1,)
    return pl.pallas_call(
        _node_lin_body,
        grid=grid,
        in_specs=[
            pl.BlockSpec((_BN1, D), lambda i: (i, 0)),
            pl.BlockSpec((D, D), lambda i: (0, 0)),
        ],
        out_specs=pl.BlockSpec((_BN1, D), lambda i: (i, 0)),
        out_shape=jax.ShapeDtypeStruct((N, D), jnp.float32),
    )(x0, wlin1)


def _split_idx_body(idx_ref, ei_ref, ej_ref):
    idx = idx_ref[...]
    ei_ref[...] = idx[0]
    ej_ref[...] = idx[1]


def _split_idx(idx):
    return pl.pallas_call(
        _split_idx_body,
        in_specs=[pl.BlockSpec((2, E), lambda: (0, 0))],
        out_specs=[
            pl.BlockSpec((E,), lambda: (0,)),
            pl.BlockSpec((E,), lambda: (0,)),
        ],
        out_shape=[
            jax.ShapeDtypeStruct((E,), jnp.int32),
            jax.ShapeDtypeStruct((E,), jnp.int32),
        ],
    )(idx)


_NC = 2
_NS = 16
_NW = _NC * _NS
_RPT = 624
_RTAIL = N - _RPT * _NS


def _sc_body_fn(ebase0, ept, ch):
    _EPT = ept
    _CH = ch
    _NFULL = _EPT // _CH
    _REM = _EPT - _NFULL * _CH
    assert _NFULL % 2 == 0 and _NFULL >= 4 and _CH % 8 == 0 and _REM % 8 == 0
    def _sc_body(x_hbm, wq_hbm, ei_hbm, ej_hbm, init_hbm, out_hbm,
                 acc, ei_v, ej_v, w_v, xg_v, ei_r, ej_r,
                 sem_in, sem_g, sem_s):
        cid = lax.axis_index("c")
        sid = lax.axis_index("s")
        pltpu.sync_copy(init_hbm.at[cid, pl.ds(sid * _RPT, _RPT)],
                        acc.at[pl.ds(sid * _RPT, _RPT)])

        @pl.when(sid == 0)
        def _seed_tail():
            pltpu.sync_copy(init_hbm.at[cid, pl.ds(_RPT * _NS, _RTAIL)],
                            acc.at[pl.ds(_RPT * _NS, _RTAIL)])
        plsc.subcore_barrier()

        ebase = ebase0 + (cid * _NS + sid) * _EPT

        def start_in(k, b):
            base = pl.multiple_of(ebase + k * _CH, 8)
            pltpu.async_copy(ei_hbm.at[pl.ds(base, _CH)], ei_v.at[b],
                             sem_in.at[b])
            pltpu.async_copy(ej_hbm.at[pl.ds(base, _CH)], ej_v.at[b],
                             sem_in.at[b])
            pltpu.async_copy(wq_hbm.at[pl.ds(base - ebase0, _CH)], w_v.at[b],
                             sem_in.at[b])

        def wait_in(k, b):
            base = pl.multiple_of(ebase + k * _CH, 8)
            pltpu.make_async_copy(ei_hbm.at[pl.ds(base, _CH)], ei_v.at[b],
                                  sem_in.at[b]).wait()
            pltpu.make_async_copy(ej_hbm.at[pl.ds(base, _CH)], ej_v.at[b],
                                  sem_in.at[b]).wait()
            pltpu.make_async_copy(wq_hbm.at[pl.ds(base - ebase0, _CH)],
                                  w_v.at[b], sem_in.at[b]).wait()

        def start_gather(b):
            pltpu.async_copy(x_hbm.at[ei_v.at[b]], xg_v.at[b], sem_g.at[b])

        def wait_gather(b):
            pltpu.make_async_copy(x_hbm.at[ei_v.at[b]], xg_v.at[b],
                                  sem_g.at[b]).wait()

        def start_scatter(b):
            pltpu.async_copy(xg_v.at[b], acc.at[ej_v.at[b]], sem_s.at[b],
                             add=True)

        def wait_scatter(b):
            pltpu.make_async_copy(xg_v.at[b], acc.at[ej_v.at[b]],
                                  sem_s.at[b]).wait()

        def compute(b):
            def rows(i, c2):
                i0 = i * 2
                i1 = i0 + 1
                for j in range(D // 16):
                    sl = pl.ds(j * 16, 16)
                    xg_v[b, i0, sl] = xg_v[b, i0, sl] * w_v[b, i0, sl]
                for j in range(D // 16):
                    sl = pl.ds(j * 16, 16)
                    xg_v[b, i1, sl] = xg_v[b, i1, sl] * w_v[b, i1, sl]
                return c2
            lax.fori_loop(0, _CH // 2, rows, 0)

        def body(k, b, first, last):
            o = 1 - b
            if not last:
                wait_in(k + 1, o)
            if not first:
                wait_scatter(o)
            if not last:
                start_gather(o)
            wait_gather(b)
            compute(b)
            start_scatter(b)
            if not last:
                start_in(k + 2, b)

        start_in(0, 0)
        start_in(1, 1)
        wait_in(0, 0)
        start_gather(0)
        body(0, 0, first=True, last=False)
        body(1, 1, first=False, last=False)

        def pair(m, carry):
            k = m * 2
            body(k, 0, first=False, last=False)
            body(k + 1, 1, first=False, last=False)
            return carry

        lax.fori_loop(1, _NFULL // 2 - 1, pair, 0)

        kl = _NFULL - 2
        wait_in(kl + 1, 1)
        wait_scatter(1)
        start_gather(1)
        wait_gather(0)
        compute(0)
        start_scatter(0)
        body(kl + 1, 1, first=False, last=True)
        wait_scatter(1)

        if _REM:
            rbase = pl.multiple_of(ebase + _NFULL * _CH, 8)
            pltpu.sync_copy(ei_hbm.at[pl.ds(rbase, _REM)], ei_r)
            pltpu.sync_copy(ej_hbm.at[pl.ds(rbase, _REM)], ej_r)
            w_r = w_v.at[0, pl.ds(0, _REM)]
            xg_r = xg_v.at[0, pl.ds(0, _REM)]
            pltpu.sync_copy(wq_hbm.at[pl.ds(rbase - ebase0, _REM)], w_r)
            pltpu.async_copy(x_hbm.at[ei_r], xg_r, sem_g.at[0]).wait()

            def rrow(i, c2):
                for j in range(D // 16):
                    sl = pl.ds(j * 16, 16)
                    xg_v[0, i, sl] = xg_v[0, i, sl] * w_v[0, i, sl]
                return c2
            lax.fori_loop(0, _REM, rrow, 0)
            pltpu.sync_copy(xg_r, acc.at[ej_r], add=True)

        plsc.subcore_barrier()
        pltpu.sync_copy(acc.at[pl.ds(sid * _RPT, _RPT)],
                        out_hbm.at[cid, pl.ds(sid * _RPT, _RPT)])

        @pl.when(sid == 0)
        def _dump_tail():
            pltpu.sync_copy(acc.at[pl.ds(_RPT * _NS, _RTAIL)],
                            out_hbm.at[cid, pl.ds(_RPT * _NS, _RTAIL)])

    return _sc_body


@functools.lru_cache(maxsize=2)
def _make_sc_kernel(ebase0, ept, ch):
    rem = ept - (ept // ch) * ch
    mesh = plsc.VectorSubcoreMesh(core_axis_name="c", subcore_axis_name="s",
                                  num_cores=_NC, num_subcores=_NS)
    return pl.kernel(
        _sc_body_fn(ebase0, ept, ch),
        out_type=jax.ShapeDtypeStruct((_NC, N, D), jnp.float32),
        mesh=mesh,
        scratch_types=[
            pltpu.VMEM_SHARED((N, D), jnp.float32),
            pltpu.VMEM((2, ch), jnp.int32),
            pltpu.VMEM((2, ch), jnp.int32),
            pltpu.VMEM((2, ch, D), jnp.float32),
            pltpu.VMEM((2, ch, D), jnp.float32),
            pltpu.VMEM((max(rem, 8),), jnp.int32),
            pltpu.VMEM((max(rem, 8),), jnp.int32),
            pltpu.SemaphoreType.DMA((2,)),
            pltpu.SemaphoreType.DMA((2,)),
            pltpu.SemaphoreType.DMA((2,)),
        ],
    )


_BN3 = 400


def _selfconn_body(x0_ref, attrs_ref, wsc2d_ref, out_ref):
    t = jnp.dot(x0_ref[...], wsc2d_ref[...], precision=_PREC,
                preferred_element_type=jnp.float32)
    attrs = attrs_ref[...]
    sc = t[:, 0:D] * attrs[:, 0:1]
    for v in range(1, A):
        sc = sc + t[:, v * D:(v + 1) * D] * attrs[:, v:v + 1]
    out_ref[...] = sc * (1.0 / math.sqrt(float(D * A)))


def _selfconn(x0, attrs, wsc2d):
    grid = (N // _BN3,)
    return pl.pallas_call(
        _selfconn_body,
        grid=grid,
        in_specs=[
            pl.BlockSpec((_BN3, D), lambda i: (i, 0)),
            pl.BlockSpec((_BN3, A), lambda i: (i, 0)),
            pl.BlockSpec((D, A * D), lambda i: (0, 0)),
        ],
        out_specs=pl.BlockSpec((_BN3, D), lambda i: (i, 0)),
        out_shape=jax.ShapeDtypeStruct((N, D), jnp.float32),
    )(x0, attrs, wsc2d)


def _final_body(agg2_ref, x0_ref, sc_ref, wlin2_ref, out_ref):
    agg = agg2_ref[0] + agg2_ref[1]
    z = jnp.dot(agg, wlin2_ref[...], precision=_PREC,
                preferred_element_type=jnp.float32) * (1.0 / math.sqrt(D))
    x0 = x0_ref[...]
    out_ref[...] = _nsilu(z + sc_ref[...]) + x0


def _final(agg2, x0, sc, wlin2):
    grid = (N // _BN3,)
    return pl.pallas_call(
        _final_body,
        grid=grid,
        in_specs=[
            pl.BlockSpec((_NC, _BN3, D), lambda i: (0, i, 0)),
            pl.BlockSpec((_BN3, D), lambda i: (i, 0)),
            pl.BlockSpec((_BN3, D), lambda i: (i, 0)),
            pl.BlockSpec((D, D), lambda i: (0, 0)),
        ],
        out_specs=pl.BlockSpec((_BN3, D), lambda i: (i, 0)),
        out_shape=jax.ShapeDtypeStruct((N, D), jnp.float32),
    )(agg2, x0, sc, wlin2)


def kernel(node_features, node_attrs, edge_attrs, edge_embedding, edge_index,
           W_lin1, W_fc1, W_fc2, W_lin2, W_sc):
    embt = edge_embedding.T
    eat = edge_attrs.T
    x = _node_linear(node_features, W_lin1)
    ei, ej = _split_idx(edge_index)
    agg = jnp.zeros((_NC, N, D), jnp.float32)
    wq = _edge_mlp_part(embt, eat, W_fc1, W_fc2, *_PARTS[0][:3])
    for p in range(len(_PARTS)):
        estart, esize, _, ch = _PARTS[p]
        nxt = None
        if p + 1 < len(_PARTS):
            nxt = _edge_mlp_part(embt, eat, W_fc1, W_fc2, *_PARTS[p + 1][:3])
        if p == len(_PARTS) - 1:
            sc = _selfconn(node_features, node_attrs, W_sc.reshape(D, A * D))
        agg = _make_sc_kernel(estart, esize // _NW, ch)(x, wq, ei, ej, agg)
        wq = nxt
    return _final(agg, node_features, sc, W_lin2)


# speedup vs baseline: 1.3300x; 1.0018x over previous
import functools
import math

import jax
import jax.numpy as jnp
import numpy as np
from jax import lax
from jax.experimental import pallas as pl
from jax.experimental.pallas import tpu as pltpu
from jax.experimental.pallas import tpu_sc as plsc

N = 10000
E = 320000
D = 128
A = 16
B = 8
H = 64

_z = np.linspace(-10.0, 10.0, 200001)
_pdf = np.exp(-0.5 * _z ** 2) / np.sqrt(2.0 * np.pi)
_silu_np = _z / (1.0 + np.exp(-_z))
_NSILU_CST = float(1.0 / np.sqrt(np.trapz(_silu_np ** 2 * _pdf, _z)))

_PREC = jax.lax.Precision.DEFAULT


def _nsilu(v):
    return jax.nn.silu(v) * _NSILU_CST


_PARTS = (
    (0, 115200, 5760, 72),
    (115200, 204800, 6400, 96),
)


def _edge_mlp_body(embt_ref, eat_ref, wfc1_ref, wfc2_ref, out_ref):
    h = jax.lax.dot_general(
        embt_ref[...], wfc1_ref[...], (((0,), (0,)), ((), ())),
        precision=_PREC, preferred_element_type=jnp.float32,
    ) * (1.0 / math.sqrt(B))
    h = _nsilu(h)
    ea_col = jax.lax.dot_general(
        eat_ref[...], jnp.ones((1, 1), jnp.float32), (((0,), (0,)), ((), ())),
        precision=_PREC, preferred_element_type=jnp.float32,
    )
    h = h * ea_col
    w = jnp.dot(h, wfc2_ref[...], precision=_PREC,
                preferred_element_type=jnp.float32) * (1.0 / math.sqrt(H))
    out_ref[...] = w


def _edge_mlp_part(embt, eat, wfc1, wfc2, estart, esize, be):
    off = estart // be
    return pl.pallas_call(
        _edge_mlp_body,
        grid=(esize // be,),
        in_specs=[
            pl.BlockSpec((B, be), lambda i: (0, i + off)),
            pl.BlockSpec((1, be), lambda i: (0, i + off)),
            pl.BlockSpec((B, H), lambda i: (0, 0)),
            pl.BlockSpec((H, D), lambda i: (0, 0)),
        ],
        out_specs=pl.BlockSpec((be, D), lambda i: (i, 0)),
        out_shape=jax.ShapeDtypeStruct((esize, D), jnp.float32),
    )(embt, eat, wfc1, wfc2)


_BN1 = 2000


def _node_lin_body(x0_ref, w_ref, out_ref):
    out_ref[...] = jnp.dot(x0_ref[...], w_ref[...], precision=_PREC,
                           preferred_element_type=jnp.float32) * (1.0 / math.sqrt(D))


def _node_linear(x0, wlin1):
    grid = (N // _BN1,)
    return pl.pallas_call(
        _node_lin_body,
        grid=grid,
        in_specs=[
            pl.BlockSpec((_BN1, D), lambda i: (i, 0)),
            pl.BlockSpec((D, D), lambda i: (0, 0)),
        ],
        out_specs=pl.BlockSpec((_BN1, D), lambda i: (i, 0)),
        out_shape=jax.ShapeDtypeStruct((N, D), jnp.float32),
    )(x0, wlin1)


def _split_idx_body(idx_ref, ei_ref, ej_ref):
    idx = idx_ref[...]
    ei_ref[...] = idx[0]
    ej_ref[...] = idx[1]


def _split_idx(idx):
    return pl.pallas_call(
        _split_idx_body,
        in_specs=[pl.BlockSpec((2, E), lambda: (0, 0))],
        out_specs=[
            pl.BlockSpec((E,), lambda: (0,)),
            pl.BlockSpec((E,), lambda: (0,)),
        ],
        out_shape=[
            jax.ShapeDtypeStruct((E,), jnp.int32),
            jax.ShapeDtypeStruct((E,), jnp.int32),
        ],
    )(idx)


_NC = 2
_NS = 16
_NW = _NC * _NS
_RPT = 624
_RTAIL = N - _RPT * _NS


def _sc_body_fn(ebase0, ept, ch):
    _EPT = ept
    _CH = ch
    _NFULL = _EPT // _CH
    _REM = _EPT - _NFULL * _CH
    assert _NFULL % 2 == 0 and _NFULL >= 4 and _CH % 8 == 0 and _REM % 8 == 0
    def _sc_body(x_hbm, wq_hbm, ei_hbm, ej_hbm, init_hbm, out_hbm,
                 acc, ei_v, ej_v, w_v, xg_v, ei_r, ej_r,
                 sem_in, sem_g, sem_s):
        cid = lax.axis_index("c")
        sid = lax.axis_index("s")
        pltpu.sync_copy(init_hbm.at[cid, pl.ds(sid * _RPT, _RPT)],
                        acc.at[pl.ds(sid * _RPT, _RPT)])

        @pl.when(sid == 0)
        def _seed_tail():
            pltpu.sync_copy(init_hbm.at[cid, pl.ds(_RPT * _NS, _RTAIL)],
                            acc.at[pl.ds(_RPT * _NS, _RTAIL)])
        plsc.subcore_barrier()

        ebase = ebase0 + (cid * _NS + sid) * _EPT

        def start_in(k, b):
            base = pl.multiple_of(ebase + k * _CH, 8)
            pltpu.async_copy(ei_hbm.at[pl.ds(base, _CH)], ei_v.at[b],
                             sem_in.at[b])
            pltpu.async_copy(ej_hbm.at[pl.ds(base, _CH)], ej_v.at[b],
                             sem_in.at[b])
            pltpu.async_copy(wq_hbm.at[pl.ds(base - ebase0, _CH)], w_v.at[b],
                             sem_in.at[b])

        def wait_in(k, b):
            base = pl.multiple_of(ebase + k * _CH, 8)
            pltpu.make_async_copy(ei_hbm.at[pl.ds(base, _CH)], ei_v.at[b],
                                  sem_in.at[b]).wait()
            pltpu.make_async_copy(ej_hbm.at[pl.ds(base, _CH)], ej_v.at[b],
                                  sem_in.at[b]).wait()
            pltpu.make_async_copy(wq_hbm.at[pl.ds(base - ebase0, _CH)],
                                  w_v.at[b], sem_in.at[b]).wait()

        def start_gather(b):
            pltpu.async_copy(x_hbm.at[ei_v.at[b]], xg_v.at[b], sem_g.at[b])

        def wait_gather(b):
            pltpu.make_async_copy(x_hbm.at[ei_v.at[b]], xg_v.at[b],
                                  sem_g.at[b]).wait()

        def start_scatter(b):
            pltpu.async_copy(xg_v.at[b], acc.at[ej_v.at[b]], sem_s.at[b],
                             add=True)

        def wait_scatter(b):
            pltpu.make_async_copy(xg_v.at[b], acc.at[ej_v.at[b]],
                                  sem_s.at[b]).wait()

        def compute(b):
            def rows(i, c2):
                i0 = i * 2
                i1 = i0 + 1
                for j in range(D // 16):
                    sl = pl.ds(j * 16, 16)
                    xg_v[b, i0, sl] = xg_v[b, i0, sl] * w_v[b, i0, sl]
                for j in range(D // 16):
                    sl = pl.ds(j * 16, 16)
                    xg_v[b, i1, sl] = xg_v[b, i1, sl] * w_v[b, i1, sl]
                return c2
            lax.fori_loop(0, _CH // 2, rows, 0)

        def body(k, b, first, last):
            o = 1 - b
            if not last:
                wait_in(k + 1, o)
            if not first:
                wait_scatter(o)
            if not last:
                start_gather(o)
            wait_gather(b)
            compute(b)
            start_scatter(b)
            if not last:
                start_in(k + 2, b)

        start_in(0, 0)
        start_in(1, 1)
        wait_in(0, 0)
        start_gather(0)
        body(0, 0, first=True, last=False)
        body(1, 1, first=False, last=False)

        def pair(m, carry):
            k = m * 2
            body(k, 0, first=False, last=False)
            body(k + 1, 1, first=False, last=False)
            return carry

        lax.fori_loop(1, _NFULL // 2 - 1, pair, 0)

        kl = _NFULL - 2
        wait_in(kl + 1, 1)
        wait_scatter(1)
        start_gather(1)
        wait_gather(0)
        compute(0)
        start_scatter(0)
        body(kl + 1, 1, first=False, last=True)
        wait_scatter(1)

        if _REM:
            rbase = pl.multiple_of(ebase + _NFULL * _CH, 8)
            pltpu.sync_copy(ei_hbm.at[pl.ds(rbase, _REM)], ei_r)
            pltpu.sync_copy(ej_hbm.at[pl.ds(rbase, _REM)], ej_r)
            w_r = w_v.at[0, pl.ds(0, _REM)]
            xg_r = xg_v.at[0, pl.ds(0, _REM)]
            pltpu.sync_copy(wq_hbm.at[pl.ds(rbase - ebase0, _REM)], w_r)
            pltpu.async_copy(x_hbm.at[ei_r], xg_r, sem_g.at[0]).wait()

            def rrow(i, c2):
                for j in range(D // 16):
                    sl = pl.ds(j * 16, 16)
                    xg_v[0, i, sl] = xg_v[0, i, sl] * w_v[0, i, sl]
                return c2
            lax.fori_loop(0, _REM, rrow, 0)
            pltpu.sync_copy(xg_r, acc.at[ej_r], add=True)

        plsc.subcore_barrier()
        pltpu.sync_copy(acc.at[pl.ds(sid * _RPT, _RPT)],
                        out_hbm.at[cid, pl.ds(sid * _RPT, _RPT)])

        @pl.when(sid == 0)
        def _dump_tail():
            pltpu.sync_copy(acc.at[pl.ds(_RPT * _NS, _RTAIL)],
                            out_hbm.at[cid, pl.ds(_RPT * _NS, _RTAIL)])

    return _sc_body


@functools.lru_cache(maxsize=2)
def _make_sc_kernel(ebase0, ept, ch):
    rem = ept - (ept // ch) * ch
    mesh = plsc.VectorSubcoreMesh(core_axis_name="c", subcore_axis_name="s",
                                  num_cores=_NC, num_subcores=_NS)
    return pl.kernel(
        _sc_body_fn(ebase0, ept, ch),
        out_type=jax.ShapeDtypeStruct((_NC, N, D), jnp.float32),
        mesh=mesh,
        scratch_types=[
            pltpu.VMEM_SHARED((N, D), jnp.float32),
            pltpu.VMEM((2, ch), jnp.int32),
            pltpu.VMEM((2, ch), jnp.int32),
            pltpu.VMEM((2, ch, D), jnp.float32),
            pltpu.VMEM((2, ch, D), jnp.float32),
            pltpu.VMEM((max(rem, 8),), jnp.int32),
            pltpu.VMEM((max(rem, 8),), jnp.int32),
            pltpu.SemaphoreType.DMA((2,)),
            pltpu.SemaphoreType.DMA((2,)),
            pltpu.SemaphoreType.DMA((2,)),
        ],
    )


_BN3 = 400


def _selfconn_body(x0_ref, attrs_ref, wsc2d_ref, out_ref):
    t = jnp.dot(x0_ref[...], wsc2d_ref[...], precision=_PREC,
                preferred_element_type=jnp.float32)
    attrs = attrs_ref[...]
    sc = t[:, 0:D] * attrs[:, 0:1]
    for v in range(1, A):
        sc = sc + t[:, v * D:(v + 1) * D] * attrs[:, v:v + 1]
    out_ref[...] = sc * (1.0 / math.sqrt(float(D * A)))


def _selfconn(x0, attrs, wsc2d):
    grid = (N // _BN3,)
    return pl.pallas_call(
        _selfconn_body,
        grid=grid,
        in_specs=[
            pl.BlockSpec((_BN3, D), lambda i: (i, 0)),
            pl.BlockSpec((_BN3, A), lambda i: (i, 0)),
            pl.BlockSpec((D, A * D), lambda i: (0, 0)),
        ],
        out_specs=pl.BlockSpec((_BN3, D), lambda i: (i, 0)),
        out_shape=jax.ShapeDtypeStruct((N, D), jnp.float32),
    )(x0, attrs, wsc2d)


def _final_body(agg2_ref, x0_ref, sc_ref, wlin2_ref, out_ref):
    agg = agg2_ref[0] + agg2_ref[1]
    z = jnp.dot(agg, wlin2_ref[...], precision=_PREC,
                preferred_element_type=jnp.float32) * (1.0 / math.sqrt(D))
    x0 = x0_ref[...]
    out_ref[...] = _nsilu(z + sc_ref[...]) + x0


def _final(agg2, x0, sc, wlin2):
    grid = (N // _BN3,)
    return pl.pallas_call(
        _final_body,
        grid=grid,
        in_specs=[
            pl.BlockSpec((_NC, _BN3, D), lambda i: (0, i, 0)),
            pl.BlockSpec((_BN3, D), lambda i: (i, 0)),
            pl.BlockSpec((_BN3, D), lambda i: (i, 0)),
            pl.BlockSpec((D, D), lambda i: (0, 0)),
        ],
        out_specs=pl.BlockSpec((_BN3, D), lambda i: (i, 0)),
        out_shape=jax.ShapeDtypeStruct((N, D), jnp.float32),
    )(agg2, x0, sc, wlin2)


def kernel(node_features, node_attrs, edge_attrs, edge_embedding, edge_index,
           W_lin1, W_fc1, W_fc2, W_lin2, W_sc):
    embt = edge_embedding.T
    eat = edge_attrs.T
    x = _node_linear(node_features, W_lin1)
    ei, ej = _split_idx(edge_index)
    agg = jnp.zeros((_NC, N, D), jnp.float32)
    wq = _edge_mlp_part(embt, eat, W_fc1, W_fc2, *_PARTS[0][:3])
    for p in range(len(_PARTS)):
        estart, esize, _, ch = _PARTS[p]
        nxt = None
        if p + 1 < len(_PARTS):
            nxt = _edge_mlp_part(embt, eat, W_fc1, W_fc2, *_PARTS[p + 1][:3])
        if p == len(_PARTS) - 1:
            sc = _selfconn(node_features, node_attrs, W_sc.reshape(D, A * D))
        agg = _make_sc_kernel(estart, esize // _NW, ch)(x, wq, ei, ej, agg)
        wq = nxt
    return _final(agg, node_features, sc, W_lin2)
